# baseline (speedup 1.0000x reference)
import functools

import jax
import jax.numpy as jnp
from jax import lax
from jax.experimental import pallas as pl
from jax.experimental.pallas import tpu as pltpu
from jax.experimental.pallas import tpu_sc as plsc

BATCH = 16384
EMB = 32
PACK = 4
LANES = PACK * EMB
TBLK = 2048
NC = 2
NS = 16
NW = NC * NS
B_PER_W = BATCH // NW
ROWS_W = B_PER_W // PACK
CHUNK = 128


def _tpack_body(t_ref, eye_ref, out_ref):
    t = lax.dot_general(t_ref[...], eye_ref[...], (((0,), (0,)), ((), ())),
                        preferred_element_type=jnp.float32)
    out_ref[...] = jnp.concatenate(
        [t[i * (TBLK // PACK):(i + 1) * (TBLK // PACK)] for i in range(PACK)],
        axis=1)


def _tc_transpose_pack(tabT, eye):
    n = tabT.shape[1]
    steps = -(-n // TBLK)
    rows = steps * (TBLK // PACK)
    return pl.pallas_call(
        _tpack_body,
        grid=(steps,),
        in_specs=[pl.BlockSpec((EMB, TBLK), lambda i: (0, i)),
                  pl.BlockSpec((EMB, EMB), lambda i: (0, 0))],
        out_specs=pl.BlockSpec((TBLK // PACK, LANES), lambda i: (i, 0)),
        out_shape=jax.ShapeDtypeStruct((rows, LANES), jnp.float32),
    )(tabT, eye)


def _gather_one_table(tab4, idx_v, blk_v, mod_v, raw_v, out_v, sem):
    def idx_body(c, _):
        ds = pl.ds(pl.multiple_of(c * 16, 16), 16)
        r = idx_v[ds]
        blk_v[ds] = lax.bitwise_or(
            lax.shift_left(lax.shift_right_logical(r, 11), 9),
            lax.bitwise_and(r, 511))
        mod_v[ds] = lax.bitwise_and(lax.shift_right_logical(r, 9), 3)
        return ()

    lax.fori_loop(0, B_PER_W // 16, idx_body, ())

    copies = []
    for c in range(B_PER_W // CHUNK):
        ds = pl.ds(c * CHUNK, CHUNK)
        copies.append(pltpu.async_copy(tab4.at[blk_v.at[ds]],
                                       raw_v.at[ds], sem))
    for cp in copies:
        cp.wait()

    lane16 = lax.iota(jnp.int32, 16)

    def sel_body(c, _):
        cb = pl.multiple_of(c * 16, 16)
        mods = mod_v[pl.ds(cb, 16)]
        for k in range(16):
            i = c * 16 + k
            rb = c * (16 // PACK) + k // PACK
            off = (k % PACK) * EMB
            lane0 = mods[k] * EMB
            rows16 = jnp.full((16,), i, jnp.int32)
            for h in range(EMB // 16):
                v = plsc.load_gather(raw_v, [rows16, lane0 + h * 16 + lane16])
                out_v[rb, pl.ds(off + h * 16, 16)] = v
        return ()

    lax.fori_loop(0, B_PER_W // 16, sel_body, ())


def _gather_body(utab4, ptab4, uidx, pidx, u_out, p_out,
                 uidx_v, pidx_v, blk_v, mod_v, raw_v, uout_v, pout_v,
                 sem_u, sem_p):
    wid = lax.axis_index("c") * NS + lax.axis_index("s")
    base = wid * B_PER_W
    pltpu.sync_copy(uidx.at[pl.ds(base, B_PER_W)], uidx_v)
    pltpu.sync_copy(pidx.at[pl.ds(base, B_PER_W)], pidx_v)

    _gather_one_table(utab4, uidx_v, blk_v, mod_v, raw_v, uout_v, sem_u)
    _gather_one_table(ptab4, pidx_v, blk_v, mod_v, raw_v, pout_v, sem_p)

    pltpu.sync_copy(uout_v, u_out.at[wid])
    pltpu.sync_copy(pout_v, p_out.at[wid])


def _sc_gather(utab4, ptab4, uidx, pidx):
    mesh = plsc.VectorSubcoreMesh(core_axis_name="c", subcore_axis_name="s")
    f = pl.kernel(
        _gather_body,
        out_type=(
            jax.ShapeDtypeStruct((NW, ROWS_W, LANES), jnp.float32),
            jax.ShapeDtypeStruct((NW, ROWS_W, LANES), jnp.float32),
        ),
        mesh=mesh,
        scratch_types=[
            pltpu.VMEM((B_PER_W,), jnp.int32),
            pltpu.VMEM((B_PER_W,), jnp.int32),
            pltpu.VMEM((B_PER_W,), jnp.int32),
            pltpu.VMEM((B_PER_W,), jnp.int32),
            pltpu.VMEM((B_PER_W, LANES), jnp.float32),
            pltpu.VMEM((ROWS_W, LANES), jnp.float32),
            pltpu.VMEM((ROWS_W, LANES), jnp.float32),
            pltpu.SemaphoreType.DMA,
            pltpu.SemaphoreType.DMA,
        ],
        compiler_params=pltpu.CompilerParams(use_tc_tiling_on_sc=False,
                                             needs_layout_passes=False),
    )
    return f(utab4, ptab4, uidx, pidx)


def _mlp_body(u_ref, p_ref, w1u_ref, w1p_ref, b1_ref, w2_ref, b2_ref,
              w3_ref, b3_ref, out_ref):
    cols = []
    for k in range(PACK):
        uk = u_ref[:, k * EMB:(k + 1) * EMB]
        pk = p_ref[:, k * EMB:(k + 1) * EMB]
        h1 = jnp.dot(uk, w1u_ref[...], preferred_element_type=jnp.float32)
        h1 += jnp.dot(pk, w1p_ref[...], preferred_element_type=jnp.float32)
        h1 = jnp.maximum(h1 + b1_ref[...], 0.0)
        h2 = jnp.dot(h1, w2_ref[...], preferred_element_type=jnp.float32)
        h2 = jnp.maximum(h2 + b2_ref[...], 0.0)
        cols.append(jnp.dot(h2, w3_ref[...],
                            preferred_element_type=jnp.float32))
    out_ref[...] = jnp.concatenate(cols, axis=1) + b3_ref[...]


def _tc_mlp(u, p, w1u_t, w1p_t, b1, w2_t, b2, w3_t, b3):
    n = BATCH // PACK
    blk = 1024
    grid = (n // blk,)
    full = lambda shape: pl.BlockSpec(shape, lambda i: (0,) * len(shape))
    return pl.pallas_call(
        _mlp_body,
        grid=grid,
        in_specs=[
            pl.BlockSpec((blk, LANES), lambda i: (i, 0)),
            pl.BlockSpec((blk, LANES), lambda i: (i, 0)),
            full((EMB, 64)),
            full((EMB, 64)),
            full((1, 64)),
            full((64, 32)),
            full((1, 32)),
            full((32, 1)),
            full((1, 1)),
        ],
        out_specs=pl.BlockSpec((blk, PACK), lambda i: (i, 0)),
        out_shape=jax.ShapeDtypeStruct((n, PACK), jnp.float32),
    )(u, p, w1u_t, w1p_t, b1, w2_t, b2, w3_t, b3)


def kernel(user_tensor, product_tensor, user_table, product_table,
           W1, b1, W2, b2, W3, b3):
    uidx = user_tensor.astype(jnp.int32)
    pidx = product_tensor.astype(jnp.int32)
    eye = jnp.eye(EMB, dtype=jnp.float32)
    utab4 = _tc_transpose_pack(user_table.T, eye)
    ptab4 = _tc_transpose_pack(product_table.T, eye)
    u_rows, p_rows = _sc_gather(utab4, ptab4, uidx, pidx)
    u = jnp.reshape(u_rows, (BATCH // PACK, LANES))
    p = jnp.reshape(p_rows, (BATCH // PACK, LANES))
    out = _tc_mlp(
        u, p,
        W1[:, :EMB].T, W1[:, EMB:].T, b1[None, :],
        W2.T, b2[None, :], W3.T, b3[None, :],
    )
    return jnp.reshape(out, (BATCH,))


# speedup vs baseline: 1.5541x; 1.5541x over previous
import functools

import jax
import jax.numpy as jnp
from jax import lax
from jax.experimental import pallas as pl
from jax.experimental.pallas import tpu as pltpu
from jax.experimental.pallas import tpu_sc as plsc

BATCH = 16384
EMB = 32
PACK = 4
LANES = PACK * EMB
TBLK = 16384
QBLK = TBLK // PACK
BSH = 14
QSH = 12
NC = 2
NS = 16
NW = NC * NS
B_PER_W = BATCH // NW
ROWS_W = B_PER_W // PACK
CHUNK = 128


def _tpack_body(t_ref, eye_ref, out_ref):
    t = lax.dot_general(t_ref[...], eye_ref[...], (((0,), (0,)), ((), ())),
                        preferred_element_type=jnp.float32)
    out_ref[...] = jnp.concatenate(
        [t[i * (TBLK // PACK):(i + 1) * (TBLK // PACK)] for i in range(PACK)],
        axis=1)


def _tc_transpose_pack(tabT, eye):
    n = tabT.shape[1]
    steps = -(-n // TBLK)
    rows = steps * (TBLK // PACK)
    return pl.pallas_call(
        _tpack_body,
        grid=(steps,),
        in_specs=[pl.BlockSpec((EMB, TBLK), lambda i: (0, i)),
                  pl.BlockSpec((EMB, EMB), lambda i: (0, 0))],
        out_specs=pl.BlockSpec((TBLK // PACK, LANES), lambda i: (i, 0)),
        out_shape=jax.ShapeDtypeStruct((rows, LANES), jnp.float32),
    )(tabT, eye)


def _gather_one_table(tab4, idx_v, blk_v, mod_v, raw_v, out_v, sem):
    def idx_body(c, _):
        ds = pl.ds(pl.multiple_of(c * 16, 16), 16)
        r = idx_v[ds]
        blk_v[ds] = lax.bitwise_or(
            lax.shift_left(lax.shift_right_logical(r, BSH), QSH),
            lax.bitwise_and(r, QBLK - 1))
        mod_v[ds] = lax.bitwise_and(lax.shift_right_logical(r, QSH), PACK - 1)
        return ()

    lax.fori_loop(0, B_PER_W // 16, idx_body, ())

    copies = []
    for c in range(B_PER_W // CHUNK):
        ds = pl.ds(c * CHUNK, CHUNK)
        copies.append(pltpu.async_copy(tab4.at[blk_v.at[ds]],
                                       raw_v.at[ds], sem))
    for cp in copies:
        cp.wait()

    lane16 = lax.iota(jnp.int32, 16)

    def sel_body(c, _):
        cb = pl.multiple_of(c * 16, 16)
        mods = mod_v[pl.ds(cb, 16)]
        for k in range(16):
            i = c * 16 + k
            rb = c * (16 // PACK) + k // PACK
            off = (k % PACK) * EMB
            lane0 = mods[k] * EMB
            rows16 = jnp.full((16,), i, jnp.int32)
            for h in range(EMB // 16):
                v = plsc.load_gather(raw_v, [rows16, lane0 + h * 16 + lane16])
                out_v[rb, pl.ds(off + h * 16, 16)] = v
        return ()

    lax.fori_loop(0, B_PER_W // 16, sel_body, ())


def _gather_body(utab4, ptab4, uidx, pidx, u_out, p_out,
                 uidx_v, pidx_v, blk_v, mod_v, raw_v, uout_v, pout_v,
                 sem_u, sem_p):
    wid = lax.axis_index("c") * NS + lax.axis_index("s")
    base = wid * B_PER_W
    pltpu.sync_copy(uidx.at[pl.ds(base, B_PER_W)], uidx_v)
    pltpu.sync_copy(pidx.at[pl.ds(base, B_PER_W)], pidx_v)

    _gather_one_table(utab4, uidx_v, blk_v, mod_v, raw_v, uout_v, sem_u)
    _gather_one_table(ptab4, pidx_v, blk_v, mod_v, raw_v, pout_v, sem_p)

    pltpu.sync_copy(uout_v, u_out.at[wid])
    pltpu.sync_copy(pout_v, p_out.at[wid])


def _sc_gather(utab4, ptab4, uidx, pidx):
    mesh = plsc.VectorSubcoreMesh(core_axis_name="c", subcore_axis_name="s")
    f = pl.kernel(
        _gather_body,
        out_type=(
            jax.ShapeDtypeStruct((NW, ROWS_W, LANES), jnp.float32),
            jax.ShapeDtypeStruct((NW, ROWS_W, LANES), jnp.float32),
        ),
        mesh=mesh,
        scratch_types=[
            pltpu.VMEM((B_PER_W,), jnp.int32),
            pltpu.VMEM((B_PER_W,), jnp.int32),
            pltpu.VMEM((B_PER_W,), jnp.int32),
            pltpu.VMEM((B_PER_W,), jnp.int32),
            pltpu.VMEM((B_PER_W, LANES), jnp.float32),
            pltpu.VMEM((ROWS_W, LANES), jnp.float32),
            pltpu.VMEM((ROWS_W, LANES), jnp.float32),
            pltpu.SemaphoreType.DMA,
            pltpu.SemaphoreType.DMA,
        ],
        compiler_params=pltpu.CompilerParams(use_tc_tiling_on_sc=False,
                                             needs_layout_passes=False),
    )
    return f(utab4, ptab4, uidx, pidx)


def _mlp_body(u_ref, p_ref, w1u_ref, w1p_ref, b1_ref, w2_ref, b2_ref,
              w3_ref, b3_ref, out_ref):
    cols = []
    for k in range(PACK):
        uk = u_ref[:, k * EMB:(k + 1) * EMB]
        pk = p_ref[:, k * EMB:(k + 1) * EMB]
        h1 = jnp.dot(uk, w1u_ref[...], preferred_element_type=jnp.float32)
        h1 += jnp.dot(pk, w1p_ref[...], preferred_element_type=jnp.float32)
        h1 = jnp.maximum(h1 + b1_ref[...], 0.0)
        h2 = jnp.dot(h1, w2_ref[...], preferred_element_type=jnp.float32)
        h2 = jnp.maximum(h2 + b2_ref[...], 0.0)
        cols.append(jnp.dot(h2, w3_ref[...],
                            preferred_element_type=jnp.float32))
    out_ref[...] = jnp.concatenate(cols, axis=1) + b3_ref[...]


def _tc_mlp(u, p, w1u_t, w1p_t, b1, w2_t, b2, w3_t, b3):
    n = BATCH // PACK
    blk = 1024
    grid = (n // blk,)
    full = lambda shape: pl.BlockSpec(shape, lambda i: (0,) * len(shape))
    return pl.pallas_call(
        _mlp_body,
        grid=grid,
        in_specs=[
            pl.BlockSpec((blk, LANES), lambda i: (i, 0)),
            pl.BlockSpec((blk, LANES), lambda i: (i, 0)),
            full((EMB, 64)),
            full((EMB, 64)),
            full((1, 64)),
            full((64, 32)),
            full((1, 32)),
            full((32, 1)),
            full((1, 1)),
        ],
        out_specs=pl.BlockSpec((blk, PACK), lambda i: (i, 0)),
        out_shape=jax.ShapeDtypeStruct((n, PACK), jnp.float32),
    )(u, p, w1u_t, w1p_t, b1, w2_t, b2, w3_t, b3)


def kernel(user_tensor, product_tensor, user_table, product_table,
           W1, b1, W2, b2, W3, b3):
    uidx = user_tensor.astype(jnp.int32)
    pidx = product_tensor.astype(jnp.int32)
    eye = jnp.eye(EMB, dtype=jnp.float32)
    utab4 = _tc_transpose_pack(user_table.T, eye)
    ptab4 = _tc_transpose_pack(product_table.T, eye)
    u_rows, p_rows = _sc_gather(utab4, ptab4, uidx, pidx)
    u = jnp.reshape(u_rows, (BATCH // PACK, LANES))
    p = jnp.reshape(p_rows, (BATCH // PACK, LANES))
    out = _tc_mlp(
        u, p,
        W1[:, :EMB].T, W1[:, EMB:].T, b1[None, :],
        W2.T, b2[None, :], W3.T, b3[None, :],
    )
    return jnp.reshape(out, (BATCH,))
